# baseline (speedup 1.0000x reference)
import jax
import jax.numpy as jnp
from jax import lax
from jax.experimental import pallas as pl
from jax.experimental.pallas import tpu as pltpu
from jax.experimental.pallas import tpu_sc as plsc

NC, NS = 2, 16
NW = NC * NS
V = 1000000
D = 64
NB = 7812
PACKED = V // 2
BPW = 245

BATCH = 4096
HIST = 50
TOTAL = BATCH * HIST
PER_W = TOTAL // NW
CHUNK = 128
K = PER_W // CHUNK
OUTP = TOTAL // 2


def _gather_body(ids_hbm, packed_hbm, out_hbm, idx_v, pidx_v,
                 gb0, gb1, gb2, ob0, ob1, gs0, gs1, gs2, os0, os1):
    gb = (gb0, gb1, gb2)
    ob = (ob0, ob1)
    gsem = (gs0, gs1, gs2)
    osem = (os0, os1)
    wid = lax.axis_index("s") * NC + lax.axis_index("c")
    obase = wid * (PER_W // 2)
    pltpu.sync_copy(ids_hbm.at[wid], idx_v)

    def mk_pidx(j, g):
        for gg in range(8):
            pidx_v[g, pl.ds(gg * 16, 16)] = (
                lax.shift_right_logical(idx_v[j, pl.ds(gg * 16, 16)], 1))

    def gather_start(j, g):
        pltpu.async_copy(packed_hbm.at[pidx_v.at[g]], gb[g], gsem[g])

    def gather_wait(j, g):
        pltpu.make_async_copy(packed_hbm.at[pidx_v.at[g]], gb[g],
                              gsem[g]).wait()

    def extract(j, g, o):
        gbf = gb[g]
        obf = ob[o]
        half = (j % 2) * 64
        iota = lax.iota(jnp.int32, 16)

        def tgrp(t, _):
            rvec = iota + 16 * t
            idxv = idx_v[j, pl.ds(16 * t, 16)]
            parv = (idxv & 1) * 64
            orowv = half + lax.shift_right_logical(rvec, 1)
            ocolv = (rvec & 1) * 64

            @plsc.parallel_loop(0, 64, unroll=4)
            def _(c):
                vals = plsc.load_gather(gbf, [rvec, parv + c])
                plsc.store_scatter(obf, [orowv, ocolv + c], vals)

            return _

        lax.fori_loop(0, 8, tgrp, None)

    def write_start(grp, o):
        pltpu.async_copy(ob[o], out_hbm.at[pl.ds(obase + grp * 128, 128)],
                         osem[o])

    def write_wait(grp, o):
        pltpu.make_async_copy(ob[o],
                              out_hbm.at[pl.ds(obase + grp * 128, 128)],
                              osem[o]).wait()

    mk_pidx(0, 0)
    gather_start(0, 0)
    mk_pidx(1, 1)
    gather_start(1, 1)
    for j in (0, 1, 2, 3):
        g = j % 3
        o = (j // 2) % 2
        if j + 2 < K:
            g2 = (j + 2) % 3
            mk_pidx(j + 2, g2)
            gather_start(j + 2, g2)
        gather_wait(j, g)
        extract(j, g, o)
        if j % 2 == 1:
            write_start(j // 2, o)

    def grp12(q, _):
        for s in range(12):
            j = 4 + 12 * q + s
            g = (4 + s) % 3
            o = ((4 + s) // 2) % 2
            g2 = (4 + s + 2) % 3
            mk_pidx(j + 2, g2)
            gather_start(j + 2, g2)
            gather_wait(j, g)
            if s % 2 == 0:
                write_wait((j - 4) // 2, o)
            extract(j, g, o)
            if s % 2 == 1:
                write_start(j // 2, o)
        return _

    lax.fori_loop(0, 3, grp12, None)

    for j in range(40, K):
        g = j % 3
        o = (j // 2) % 2
        if j + 2 < K:
            g2 = (j + 2) % 3
            mk_pidx(j + 2, g2)
            gather_start(j + 2, g2)
        gather_wait(j, g)
        if j % 2 == 0:
            write_wait((j - 4) // 2, o)
        extract(j, g, o)
        if j % 2 == 1:
            write_start(j // 2, o)

    write_wait(23, 1)
    write_wait(24, 0)


def _gather(ids3, packed):
    mesh = plsc.VectorSubcoreMesh(core_axis_name="c", subcore_axis_name="s")
    run = pl.kernel(
        _gather_body,
        out_type=jax.ShapeDtypeStruct((OUTP, 128), jnp.float32),
        mesh=mesh,
        scratch_types=[
            pltpu.VMEM((K, CHUNK), jnp.int32),
            pltpu.VMEM((3, CHUNK), jnp.int32),
            pltpu.VMEM((CHUNK, 128), jnp.float32),
            pltpu.VMEM((CHUNK, 128), jnp.float32),
            pltpu.VMEM((CHUNK, 128), jnp.float32),
            pltpu.VMEM((CHUNK, 128), jnp.float32),
            pltpu.VMEM((CHUNK, 128), jnp.float32),
            pltpu.SemaphoreType.DMA,
            pltpu.SemaphoreType.DMA,
            pltpu.SemaphoreType.DMA,
            pltpu.SemaphoreType.DMA,
            pltpu.SemaphoreType.DMA,
        ],
        compiler_params=pltpu.CompilerParams(use_tc_tiling_on_sc=True,
                                             needs_layout_passes=False,
                                             disable_bounds_checks=True,
                                             disable_semaphore_checks=True),
    )
    return run(ids3, packed)


@jax.jit
def _fused(input_ids, table):
    packed = jnp.concatenate([table[0::2], table[1::2]], axis=1)
    ids3 = input_ids.astype(jnp.int32).reshape(NW, K, CHUNK)
    out2 = _gather(ids3, packed)
    return out2.reshape(BATCH, HIST, D)


def kernel(input_ids, embed_tokens_weight):
    return _fused(input_ids, embed_tokens_weight)


# speedup vs baseline: 14.0737x; 14.0737x over previous
import jax
import jax.numpy as jnp
from jax import lax
from jax.experimental import pallas as pl
from jax.experimental.pallas import tpu as pltpu
from jax.experimental.pallas import tpu_sc as plsc

NC, NS = 2, 16
NW = NC * NS
V = 1000000
D = 64
NB = 7812
PACKED = V // 2
BPW = 245

BATCH = 4096
HIST = 50
TOTAL = BATCH * HIST
PER_W = TOTAL // NW
CHUNK = 128
K = PER_W // CHUNK
OUTP = TOTAL // 2


def _relayout_body(tabT_hbm, remT_hbm, packed_hbm, tb0, tb1, pb0, pb1,
                   gs0, gs1, os0, os1):
    tb = (tb0, tb1)
    pb = (pb0, pb1)
    gsem = (gs0, gs1)
    osem = (os0, os1)
    wid = lax.axis_index("s") * NC + lax.axis_index("c")
    wstart = wid * BPW
    nblk = jnp.minimum(wstart + BPW, NB) - wstart

    def load_start(i, s):
        pltpu.async_copy(tabT_hbm.at[:, pl.ds((wstart + i) * 128, 128)],
                         tb[s], gsem[s])

    def load_wait(i, s):
        pltpu.make_async_copy(tabT_hbm.at[:, pl.ds((wstart + i) * 128, 128)],
                              tb[s], gsem[s]).wait()

    def store_start(i, s):
        pltpu.async_copy(pb[s], packed_hbm.at[pl.ds((wstart + i) * 64, 64)],
                         osem[s])

    def store_wait(i, s):
        pltpu.make_async_copy(pb[s],
                              packed_hbm.at[pl.ds((wstart + i) * 64, 64)],
                              osem[s]).wait()

    def transpose(s, nrows):
        tbf = tb[s]
        pbf = pb[s]
        iota = lax.iota(jnp.int32, 16)

        @plsc.parallel_loop(0, 0, unroll=2)
        def _(j):
            col = jnp.zeros((16,), jnp.int32) + 2 * j
            for g in range(4):
                rows_g = iota + 16 * g
                lo = plsc.load_gather(tbf, [rows_g, col])
                hi = plsc.load_gather(tbf, [rows_g, col + 1])
                pbf[j, pl.ds(g * 16, 16)] = lo
                pbf[j, pl.ds(64 + g * 16, 16)] = hi

    load_start(0, 0)
    load_wait(0, 0)

    @pl.when(nblk > 1)
    def _():
        load_start(1, 1)

    transpose(0, 64)
    store_start(0, 0)

    def pair(q, _):
        for s, ioff in ((1, 1), (0, 2)):
            i = 2 * q + ioff

            @pl.when(i < nblk)
            def _():
                load_wait(i, s)

                @pl.when(i + 1 < nblk)
                def _():
                    load_start(i + 1, 1 - s)

                transpose(s, 64)
                store_wait(i - 1, 1 - s)
                store_start(i, s)

        return _

    lax.fori_loop(0, (BPW - 1) // 2, pair, None)
    store_wait(nblk - 1, 0)

    @pl.when(wid == NW - 1)
    def _():
        pltpu.sync_copy(remT_hbm, tb[0])
        transpose(0, 32)
        pltpu.sync_copy(pb[0].at[pl.ds(0, 32)],
                        packed_hbm.at[pl.ds(NB * 64, 32)])


def _relayout(tabT, remT):
    mesh = plsc.VectorSubcoreMesh(core_axis_name="c", subcore_axis_name="s")
    run = pl.kernel(
        _relayout_body,
        out_type=jax.ShapeDtypeStruct((PACKED, 128), jnp.float32),
        mesh=mesh,
        scratch_types=[
            pltpu.VMEM((D, 128), jnp.float32),
            pltpu.VMEM((D, 128), jnp.float32),
            pltpu.VMEM((D, 128), jnp.float32),
            pltpu.VMEM((D, 128), jnp.float32),
            pltpu.SemaphoreType.DMA,
            pltpu.SemaphoreType.DMA,
            pltpu.SemaphoreType.DMA,
            pltpu.SemaphoreType.DMA,
        ],
        compiler_params=pltpu.CompilerParams(use_tc_tiling_on_sc=True,
                                             needs_layout_passes=False,
                                             disable_bounds_checks=True,
                                             disable_semaphore_checks=True),
    )
    return run(tabT, remT)


def _gather_body(ids_hbm, packed_hbm, out_hbm, idx_v, pidx_v,
                 gb0, gb1, gb2, ob0, ob1, gs0, gs1, gs2, os0, os1):
    gb = (gb0, gb1, gb2)
    ob = (ob0, ob1)
    gsem = (gs0, gs1, gs2)
    osem = (os0, os1)
    wid = lax.axis_index("s") * NC + lax.axis_index("c")
    obase = wid * (PER_W // 2)
    pltpu.sync_copy(ids_hbm.at[wid], idx_v)

    def mk_pidx(j, g):
        for gg in range(8):
            pidx_v[g, pl.ds(gg * 16, 16)] = (
                lax.shift_right_logical(idx_v[j, pl.ds(gg * 16, 16)], 1))

    def gather_start(j, g):
        pltpu.async_copy(packed_hbm.at[pidx_v.at[g]], gb[g], gsem[g])

    def gather_wait(j, g):
        pltpu.make_async_copy(packed_hbm.at[pidx_v.at[g]], gb[g],
                              gsem[g]).wait()

    def extract(j, g, o):
        gbf = gb[g]
        obf = ob[o]
        half = (j % 2) * 64
        iota = lax.iota(jnp.int32, 16)

        def tgrp(t, _):
            rvec = iota + 16 * t
            idxv = idx_v[j, pl.ds(16 * t, 16)]
            parv = (idxv & 1) * 64
            orowv = half + lax.shift_right_logical(rvec, 1)
            ocolv = (rvec & 1) * 64

            @plsc.parallel_loop(0, 64, unroll=4)
            def _(c):
                vals = plsc.load_gather(gbf, [rvec, parv + c])
                plsc.store_scatter(obf, [orowv, ocolv + c], vals)

            return _

        lax.fori_loop(0, 8, tgrp, None)

    def write_start(grp, o):
        pltpu.async_copy(ob[o], out_hbm.at[pl.ds(obase + grp * 128, 128)],
                         osem[o])

    def write_wait(grp, o):
        pltpu.make_async_copy(ob[o],
                              out_hbm.at[pl.ds(obase + grp * 128, 128)],
                              osem[o]).wait()

    mk_pidx(0, 0)
    gather_start(0, 0)
    mk_pidx(1, 1)
    gather_start(1, 1)
    for j in (0, 1, 2, 3):
        g = j % 3
        o = (j // 2) % 2
        if j + 2 < K:
            g2 = (j + 2) % 3
            mk_pidx(j + 2, g2)
            gather_start(j + 2, g2)
        gather_wait(j, g)
        extract(j, g, o)
        if j % 2 == 1:
            write_start(j // 2, o)

    def grp12(q, _):
        for s in range(12):
            j = 4 + 12 * q + s
            g = (4 + s) % 3
            o = ((4 + s) // 2) % 2
            g2 = (4 + s + 2) % 3
            mk_pidx(j + 2, g2)
            gather_start(j + 2, g2)
            gather_wait(j, g)
            if s % 2 == 0:
                write_wait((j - 4) // 2, o)
            extract(j, g, o)
            if s % 2 == 1:
                write_start(j // 2, o)
        return _

    lax.fori_loop(0, 3, grp12, None)

    for j in range(40, K):
        g = j % 3
        o = (j // 2) % 2
        if j + 2 < K:
            g2 = (j + 2) % 3
            mk_pidx(j + 2, g2)
            gather_start(j + 2, g2)
        gather_wait(j, g)
        if j % 2 == 0:
            write_wait((j - 4) // 2, o)
        extract(j, g, o)
        if j % 2 == 1:
            write_start(j // 2, o)

    write_wait(23, 1)
    write_wait(24, 0)


def _gather(ids3, packed):
    mesh = plsc.VectorSubcoreMesh(core_axis_name="c", subcore_axis_name="s")
    run = pl.kernel(
        _gather_body,
        out_type=jax.ShapeDtypeStruct((OUTP, 128), jnp.float32),
        mesh=mesh,
        scratch_types=[
            pltpu.VMEM((K, CHUNK), jnp.int32),
            pltpu.VMEM((3, CHUNK), jnp.int32),
            pltpu.VMEM((CHUNK, 128), jnp.float32),
            pltpu.VMEM((CHUNK, 128), jnp.float32),
            pltpu.VMEM((CHUNK, 128), jnp.float32),
            pltpu.VMEM((CHUNK, 128), jnp.float32),
            pltpu.VMEM((CHUNK, 128), jnp.float32),
            pltpu.SemaphoreType.DMA,
            pltpu.SemaphoreType.DMA,
            pltpu.SemaphoreType.DMA,
            pltpu.SemaphoreType.DMA,
            pltpu.SemaphoreType.DMA,
        ],
        compiler_params=pltpu.CompilerParams(use_tc_tiling_on_sc=True,
                                             needs_layout_passes=False,
                                             disable_bounds_checks=True,
                                             disable_semaphore_checks=True),
    )
    return run(ids3, packed)


@jax.jit
def _fused(input_ids, table):
    tabT = table.T
    remT = jnp.pad(lax.slice(tabT, (0, NB * 128), (D, V)), ((0, 0), (0, 64)))
    packed = _relayout(tabT, remT)
    ids3 = input_ids.astype(jnp.int32).reshape(NW, K, CHUNK)
    out2 = _gather(ids3, packed)
    return out2.reshape(BATCH, HIST, D)


def kernel(input_ids, embed_tokens_weight):
    return _fused(input_ids, embed_tokens_weight)
